# baseline (speedup 1.0000x reference)
import jax
import jax.numpy as jnp
from jax import lax
from jax.experimental import pallas as pl

_B, _C, _H, _W = 8, 21, 512, 512
_HW = _H * _W
_EPS = 1e-05
_CHUNK = 32768
_NCHUNK = _HW // _CHUNK


def _dice_sums_body(pred_ref, tgt_ref, out_ref):
    j = pl.program_id(1)
    p = pred_ref[0]
    t = tgt_ref[0]
    cls = lax.broadcasted_iota(jnp.int32, (_C, _CHUNK), 0)
    m = (cls == t).astype(jnp.float32)
    inter = jnp.sum(p * m, axis=1, keepdims=True)
    p2 = jnp.sum(p * p, axis=1, keepdims=True)
    cnt = jnp.sum(m, axis=1, keepdims=True)
    part = jnp.concatenate([inter, p2, cnt], axis=0)

    @pl.when(j == 0)
    def _():
        out_ref[0] = part

    @pl.when(j != 0)
    def _():
        out_ref[0] += part


def kernel(prediction, target):
    pred3 = prediction.reshape(_B, _C, _HW)
    tgt3 = target.astype(jnp.int32).reshape(_B, 1, _HW)

    sums = pl.pallas_call(
        _dice_sums_body,
        grid=(_B, _NCHUNK),
        in_specs=[
            pl.BlockSpec((1, _C, _CHUNK), lambda b, j: (b, 0, j)),
            pl.BlockSpec((1, 1, _CHUNK), lambda b, j: (b, 0, j)),
        ],
        out_specs=pl.BlockSpec((1, 3 * _C, 1), lambda b, j: (b, 0, 0)),
        out_shape=jax.ShapeDtypeStruct((_B, 3 * _C, 1), jnp.float32),
    )(pred3, tgt3)

    sums = sums[..., 0]
    inter = sums[:, :_C]
    p2 = sums[:, _C:2 * _C]
    cnt = sums[:, 2 * _C:]
    dice = (2.0 * inter + _EPS) / (p2 + cnt + _EPS)
    return 1.0 - dice.mean()


# speedup vs baseline: 1.2645x; 1.2645x over previous
import jax
import jax.numpy as jnp
from jax import lax
from jax.experimental import pallas as pl

_B, _C, _H, _W = 8, 21, 512, 512
_HW = _H * _W
_EPS = 1e-05
_K = 256
_CHUNK = _K * 128
_NCHUNK = _HW // _CHUNK


def _dice_sums_body(pred_ref, tgt_ref, out_ref):
    j = pl.program_id(1)
    p = pred_ref[0]
    t = tgt_ref[0]
    cls = lax.broadcasted_iota(jnp.int32, (_C, 1, 1), 0)
    m = (cls == t).astype(jnp.float32)
    inter = jnp.sum(p * m, axis=1)
    p2 = jnp.sum(p * p, axis=1)
    cnt = jnp.sum(m, axis=1)
    part = jnp.concatenate([inter, p2, cnt], axis=0)

    @pl.when(j == 0)
    def _():
        out_ref[0] = part

    @pl.when(j != 0)
    def _():
        out_ref[0] += part


def kernel(prediction, target):
    pred4 = prediction.reshape(_B, _C, _HW // 128, 128)
    tgt4 = target.astype(jnp.int32).reshape(_B, 1, _HW // 128, 128)

    sums = pl.pallas_call(
        _dice_sums_body,
        grid=(_B, _NCHUNK),
        in_specs=[
            pl.BlockSpec((1, _C, _K, 128), lambda b, j: (b, 0, j, 0)),
            pl.BlockSpec((1, 1, _K, 128), lambda b, j: (b, 0, j, 0)),
        ],
        out_specs=pl.BlockSpec((1, 3 * _C, 128), lambda b, j: (b, 0, 0)),
        out_shape=jax.ShapeDtypeStruct((_B, 3 * _C, 128), jnp.float32),
    )(pred4, tgt4)

    sums = sums.sum(axis=-1)
    inter = sums[:, :_C]
    p2 = sums[:, _C:2 * _C]
    cnt = sums[:, 2 * _C:]
    dice = (2.0 * inter + _EPS) / (p2 + cnt + _EPS)
    return 1.0 - dice.mean()
